# baseline (speedup 1.0000x reference)
import functools

import jax
import jax.numpy as jnp
from jax import lax
from jax.experimental import pallas as pl
from jax.experimental.pallas import tpu as pltpu
from jax.experimental.pallas import tpu_sc as plsc

_NC = 2
_NS = 16
_NW = _NC * _NS
_LANES = 16

_B = 4
_S = 4096
_E = 1024
_R = 8
_TILE = _R * _E
_ROWS_PER_W = _S // _NW
_T = _ROWS_PER_W // _R


def _sc_body(tok_hbm, pos_hbm, out_hbm, pos_v, t0, t1, t2, t3):
    wid = lax.axis_index("s") * _NC + lax.axis_index("c")
    s0 = wid * _ROWS_PER_W
    bufs = (t0, t1, t2, t3)

    @pl.loop(0, _T)
    def _tile_loop(t):
        base = (s0 + t * _R) * _E
        pltpu.sync_copy(pos_hbm.at[pl.ds(base, _TILE)], pos_v)
        for b in range(_B):
            pltpu.sync_copy(tok_hbm.at[pl.ds(b * _S * _E + base, _TILE)],
                            bufs[b])

        @pl.loop(0, _TILE, step=_LANES)
        def _add_loop(i):
            p = pos_v[pl.ds(i, _LANES)]
            for b in range(_B):
                bufs[b][pl.ds(i, _LANES)] += p

        for b in range(_B):
            pltpu.sync_copy(bufs[b],
                            out_hbm.at[pl.ds(b * _S * _E + base, _TILE)])


def kernel(token_embedding, pos_table):
    B, S, E = token_embedding.shape
    tok_flat = token_embedding.reshape(-1)
    pos_flat = pos_table.reshape(-1)

    mesh = plsc.VectorSubcoreMesh(core_axis_name="c", subcore_axis_name="s")
    run = functools.partial(
        pl.kernel,
        out_type=jax.ShapeDtypeStruct((B * S * E,), jnp.float32),
        mesh=mesh,
        scratch_types=[pltpu.VMEM((_TILE,), jnp.float32)] * 5,
    )(_sc_body)
    out_flat = run(tok_flat, pos_flat)
    return out_flat.reshape(B, S, E)


# speedup vs baseline: 1.3651x; 1.3651x over previous
import functools

import jax
import jax.numpy as jnp
from jax import lax
from jax.experimental import pallas as pl
from jax.experimental.pallas import tpu as pltpu
from jax.experimental.pallas import tpu_sc as plsc

_NC = 2
_NS = 16
_NW = _NC * _NS
_LANES = 16
_UNROLL = 4

_B = 4
_S = 4096
_E = 1024
_R = 8
_TILE = _R * _E
_ROWS_PER_W = _S // _NW
_T = _ROWS_PER_W // _R


def _sc_body(tok_hbm, pos_hbm, out_hbm,
             pos0, pos1,
             a0, a1, a2, a3, b0, b1, b2, b3, c0, c1, c2, c3,
             isem0, isem1, osem0, osem1):
    wid = lax.axis_index("s") * _NC + lax.axis_index("c")
    s0e = wid * _ROWS_PER_W * _E
    pos_bufs = (pos0, pos1)
    tok_sets = ((a0, a1, a2, a3), (b0, b1, b2, b3), (c0, c1, c2, c3))
    isems = (isem0, isem1)
    osems = (osem0, osem1)

    def issue_in(t):
        base = s0e + t * _TILE
        sem = isems[t % 2]
        tset = tok_sets[t % 3]
        hs = [pltpu.async_copy(pos_hbm.at[pl.ds(base, _TILE)],
                               pos_bufs[t % 2], sem)]
        for b in range(_B):
            hs.append(pltpu.async_copy(
                tok_hbm.at[pl.ds(b * _S * _E + base, _TILE)], tset[b], sem))
        return hs

    def issue_out(t):
        base = s0e + t * _TILE
        sem = osems[t % 2]
        tset = tok_sets[t % 3]
        return [pltpu.async_copy(
            tset[b], out_hbm.at[pl.ds(b * _S * _E + base, _TILE)], sem)
            for b in range(_B)]

    def compute(t):
        pos_v = pos_bufs[t % 2]
        tset = tok_sets[t % 3]

        @pl.loop(0, _TILE, step=_LANES * _UNROLL)
        def _add_loop(i):
            for u in range(_UNROLL):
                sl = pl.ds(i + u * _LANES, _LANES)
                p = pos_v[sl]
                for b in range(_B):
                    plsc.addupdate(tset[b].at[sl], p)

    in_h = {0: issue_in(0)}
    out_h = {}
    for t in range(_T):
        if t >= 2:
            for h in out_h[t - 2]:
                h.wait()
        if t + 1 < _T:
            in_h[t + 1] = issue_in(t + 1)
        for h in in_h[t]:
            h.wait()
        compute(t)
        out_h[t] = issue_out(t)
    for h in out_h[_T - 2]:
        h.wait()
    for h in out_h[_T - 1]:
        h.wait()


def kernel(token_embedding, pos_table):
    B, S, E = token_embedding.shape
    tok_flat = token_embedding.reshape(-1)
    pos_flat = pos_table.reshape(-1)

    mesh = plsc.VectorSubcoreMesh(core_axis_name="c", subcore_axis_name="s")
    run = functools.partial(
        pl.kernel,
        out_type=jax.ShapeDtypeStruct((B * S * E,), jnp.float32),
        mesh=mesh,
        scratch_types=(
            [pltpu.VMEM((_TILE,), jnp.float32)] * 14
            + [pltpu.SemaphoreType.DMA] * 4
        ),
    )(_sc_body)
    out_flat = run(tok_flat, pos_flat)
    return out_flat.reshape(B, S, E)


# speedup vs baseline: 3.8807x; 2.8429x over previous
import functools

import jax
import jax.numpy as jnp
from jax import lax
from jax.experimental import pallas as pl
from jax.experimental.pallas import tpu as pltpu
from jax.experimental.pallas import tpu_sc as plsc

_NC = 2
_NS = 16
_NW = _NC * _NS
_LANES = 16
_UNROLL = 4

_B = 4
_S = 4096
_E = 1024
_R = 8
_ROWS_PER_W = _S // _NW
_T = _ROWS_PER_W // _R


def _sc_body(tok_hbm, pos_hbm, out_hbm,
             pos0, pos1,
             a0, a1, a2, a3, b0, b1, b2, b3, c0, c1, c2, c3,
             isem0, isem1, osem0, osem1):
    wid = lax.axis_index("s") * _NC + lax.axis_index("c")
    s0 = wid * _ROWS_PER_W
    pos_bufs = (pos0, pos1)
    tok_sets = ((a0, a1, a2, a3), (b0, b1, b2, b3), (c0, c1, c2, c3))
    isems = (isem0, isem1)
    osems = (osem0, osem1)

    def issue_in(t):
        row = s0 + t * _R
        sem = isems[t % 2]
        tset = tok_sets[t % 3]
        hs = [pltpu.async_copy(pos_hbm.at[pl.ds(row, _R), :],
                               pos_bufs[t % 2], sem)]
        for b in range(_B):
            hs.append(pltpu.async_copy(
                tok_hbm.at[b, pl.ds(row, _R), :], tset[b], sem))
        return hs

    def issue_out(t):
        row = s0 + t * _R
        sem = osems[t % 2]
        tset = tok_sets[t % 3]
        return [pltpu.async_copy(
            tset[b], out_hbm.at[b, pl.ds(row, _R), :], sem)
            for b in range(_B)]

    def compute(t):
        pos_v = pos_bufs[t % 2]
        tset = tok_sets[t % 3]

        @pl.loop(0, _R)
        def _row_loop(r):
            @pl.loop(0, _E, step=_LANES * _UNROLL)
            def _add_loop(c):
                for u in range(_UNROLL):
                    sl = pl.ds(c + u * _LANES, _LANES)
                    p = pos_v[r, sl]
                    for b in range(_B):
                        plsc.addupdate(tset[b].at[r, sl], p)

    in_h = {0: issue_in(0)}
    out_h = {}
    for t in range(_T):
        if t >= 2:
            for h in out_h[t - 2]:
                h.wait()
        if t + 1 < _T:
            in_h[t + 1] = issue_in(t + 1)
        for h in in_h[t]:
            h.wait()
        compute(t)
        out_h[t] = issue_out(t)
    for h in out_h[_T - 2]:
        h.wait()
    for h in out_h[_T - 1]:
        h.wait()


def kernel(token_embedding, pos_table):
    B, S, E = token_embedding.shape

    mesh = plsc.VectorSubcoreMesh(core_axis_name="c", subcore_axis_name="s")
    run = functools.partial(
        pl.kernel,
        out_type=jax.ShapeDtypeStruct((B, S, E), jnp.float32),
        mesh=mesh,
        scratch_types=(
            [pltpu.VMEM((_R, _E), jnp.float32)] * 14
            + [pltpu.SemaphoreType.DMA] * 4
        ),
    )(_sc_body)
    return run(token_embedding, pos_table)


# speedup vs baseline: 3.8943x; 1.0035x over previous
import functools

import jax
import jax.numpy as jnp
from jax import lax
from jax.experimental import pallas as pl
from jax.experimental.pallas import tpu as pltpu
from jax.experimental.pallas import tpu_sc as plsc

_NC = 2
_NS = 16
_NW = _NC * _NS
_LANES = 16
_UNROLL = 4

_B = 4
_S = 4096
_E = 1024
_R = 8
_ROWS_PER_W = _S // _NW
_T = _ROWS_PER_W // _R


def _sc_body(tok_hbm, pos_hbm, out_hbm,
             pos0, pos1, tok0, tok1, tok2,
             isem0, isem1, osem0, osem1):
    wid = lax.axis_index("s") * _NC + lax.axis_index("c")
    s0 = wid * _ROWS_PER_W
    pos_bufs = (pos0, pos1)
    tok_sets = (tok0, tok1, tok2)
    isems = (isem0, isem1)
    osems = (osem0, osem1)

    def issue_in(t):
        row = s0 + t * _R
        sem = isems[t % 2]
        tset = tok_sets[t % 3]
        return [
            pltpu.async_copy(pos_hbm.at[pl.ds(row, _R), :],
                             pos_bufs[t % 2], sem),
            pltpu.async_copy(tok_hbm.at[:, pl.ds(row, _R), :], tset, sem),
        ]

    def issue_out(t):
        row = s0 + t * _R
        tset = tok_sets[t % 3]
        return [pltpu.async_copy(
            tset, out_hbm.at[:, pl.ds(row, _R), :], osems[t % 2])]

    def compute(t):
        pos_v = pos_bufs[t % 2]
        tset = tok_sets[t % 3]

        @pl.loop(0, _R)
        def _row_loop(r):
            @pl.loop(0, _E, step=_LANES * _UNROLL)
            def _add_loop(c):
                for u in range(_UNROLL):
                    sl = pl.ds(c + u * _LANES, _LANES)
                    p = pos_v[r, sl]
                    for b in range(_B):
                        plsc.addupdate(tset.at[b, r, sl], p)

    in_h = {0: issue_in(0)}
    out_h = {}
    for t in range(_T):
        if t >= 2:
            for h in out_h[t - 2]:
                h.wait()
        if t + 1 < _T:
            in_h[t + 1] = issue_in(t + 1)
        for h in in_h[t]:
            h.wait()
        compute(t)
        out_h[t] = issue_out(t)
    for h in out_h[_T - 2]:
        h.wait()
    for h in out_h[_T - 1]:
        h.wait()


def kernel(token_embedding, pos_table):
    B, S, E = token_embedding.shape

    mesh = plsc.VectorSubcoreMesh(core_axis_name="c", subcore_axis_name="s")
    run = functools.partial(
        pl.kernel,
        out_type=jax.ShapeDtypeStruct((B, S, E), jnp.float32),
        mesh=mesh,
        scratch_types=(
            [pltpu.VMEM((_R, _E), jnp.float32)] * 2
            + [pltpu.VMEM((_B, _R, _E), jnp.float32)] * 3
            + [pltpu.SemaphoreType.DMA] * 4
        ),
    )(_sc_body)
    return run(token_embedding, pos_table)
